# baseline (speedup 1.0000x reference)
import functools

import jax
import jax.numpy as jnp
from jax import lax
from jax.experimental import pallas as pl
from jax.experimental.pallas import tpu as pltpu
from jax.experimental.pallas import tpu_sc as plsc

_CHUNK = 128


@functools.lru_cache(maxsize=None)
def _build_gather(B: int, W: int):
    info = plsc.get_sparse_core_info()
    NC, NS, L = info.num_cores, info.num_subcores, info.num_lanes
    NW = NC * NS
    assert B % (NW * L) == 0
    bpw = B // NW
    nch = bpw // _CHUNK
    assert bpw % _CHUNK == 0
    mesh = plsc.VectorSubcoreMesh(core_axis_name="c", subcore_axis_name="s")

    @functools.partial(
        pl.kernel,
        mesh=mesh,
        out_type=jax.ShapeDtypeStruct((NW, nch, _CHUNK), jnp.float32),
        scratch_types=[
            pltpu.VMEM((bpw,), jnp.int32),
            pltpu.VMEM((bpw,), jnp.int32),
            pltpu.VMEM((nch, _CHUNK), jnp.int32),
            pltpu.VMEM((nch, _CHUNK), jnp.float32),
            pltpu.SemaphoreType.DMA,
        ],
    )
    def body(rows_hbm, cols_hbm, table_hbm, out_hbm,
             rows_v, cols_v, idx_v, val_v, sem):
        wid = lax.axis_index("s") * NC + lax.axis_index("c")
        base = wid * bpw
        pltpu.sync_copy(rows_hbm.at[pl.ds(base, bpw)], rows_v)
        pltpu.sync_copy(cols_hbm.at[pl.ds(base, bpw)], cols_v)
        per_chunk = _CHUNK // L
        for i in range(bpw // L):
            r = rows_v[pl.ds(i * L, L)]
            c = cols_v[pl.ds(i * L, L)]
            idx_v[i // per_chunk, pl.ds((i % per_chunk) * L, L)] = r * W + c
        copies = [
            pltpu.async_copy(table_hbm.at[idx_v.at[j]], val_v.at[j], sem)
            for j in range(nch)
        ]
        for cp in copies:
            cp.wait()
        pltpu.sync_copy(val_v, out_hbm.at[wid])

    return body


def kernel(obs, acs, cost_matrix):
    del acs
    B = obs.shape[0]
    H, W = cost_matrix.shape
    obs32 = obs.astype(jnp.int32)
    rows = obs32[:, 0]
    cols = obs32[:, 1]
    table = cost_matrix.reshape(H * W)
    out = _build_gather(B, W)(rows, cols, table)
    return out.reshape(B)
